# Initial kernel scaffold; baseline (speedup 1.0000x reference)
import jax
import jax.numpy as jnp
from jax.experimental import pallas as pl


def kernel(feat_crew, feat_plane, feat_state, feat_value, edge_c_in, edge_repairing, edge_p_in, edge_p_to, edge_repaired_by, edge_s_in, edge_s_to, edge_v_to, W_crew, b_crew, W_plane, b_plane, W_c_in, b_c_in, W_repairing, b_repairing, W_p_in, b_p_in, W_p_to, b_p_to, W_repaired_by, b_repaired_by, W_s_in, b_s_in, W_s_to, b_s_to, W_v_to, b_v_to, c_in_src, c_in_dst, p_in_src, p_in_dst):
    raise NotImplementedError("write your pallas kernel here")


# speedup vs baseline: 46.1085x; 46.1085x over previous
import jax
import jax.numpy as jnp
from jax import lax
from jax.experimental import pallas as pl
from jax.experimental.pallas import tpu as pltpu
from jax.experimental.pallas import tpu_sc as plsc

H = 4
D = 32
N = 50000
E = 500000
DIN = 128
HD = H * D

NT = 16
WINE = 256
NWIN = 124
EPT = WINE * NWIN
EP = EPT * NT
EPR = EP // 128
NACC = 50048
DLEN = 50176
ZF = DLEN // NT


def _splat(vec, lane):
    idx = jnp.full((16,), lane, dtype=jnp.int32)
    return jnp.take_along_axis(vec, idx, axis=0)


def _sc_edge_kernel(tblW, asW, adW, tblU, edgeW, edgeU,
                    ftW, wW, denW, ftU,
                    src2, dst2, gix, rows, srow, drow, whm, w2,
                    zrow, zflat, numer, denom, dsem):
    c = lax.axis_index("c")
    t = lax.axis_index("s")

    lane = lax.iota(jnp.int32, 16)
    epat = lane // 4
    hpat = lane % 4

    zv = jnp.zeros((16,), jnp.float32)

    @pl.loop(0, 256)
    def _(i):
        zrow[i // 2, pl.ds((i % 2) * 16, 16)] = zv

    @pl.loop(0, ZF // 16)
    def _(i):
        zflat[pl.ds(i * 16, 16)] = zv

    def zero_numer():
        for k in range(25):
            ch = t + 16 * k

            @pl.when(ch < 391)
            def _():
                nb = pl.multiple_of(ch * 128, 128)
                pltpu.sync_copy(zrow, numer.at[pl.ds(nb, 128)])

    def zero_denom():
        pltpu.sync_copy(zflat,
                        denom.at[pl.ds(pl.multiple_of(t * ZF, ZF), ZF)])

    zero_numer()
    plsc.subcore_barrier()

    @pl.loop(0, NWIN)
    def _(w):
        base = pl.multiple_of(t * EPT + w * WINE, 256)
        rb = pl.multiple_of(base // 128, 2)
        pltpu.sync_copy(edgeW.at[c, 0, pl.ds(rb, 2)], src2)
        pltpu.sync_copy(edgeW.at[c, 1, pl.ds(rb, 2)], dst2)

        @pl.loop(0, 2)
        def _(r):
            for g in range(8):
                gix[r, pl.ds(g * 16, 16)] = (
                    src2[r, pl.ds(g * 16, 16)] + c * N)
                dst2[r, pl.ds(g * 16, 16)] = jnp.minimum(
                    dst2[r, pl.ds(g * 16, 16)], N - 1) + c * N

        for j in range(2):
            pltpu.async_copy(asW.at[gix.at[j]],
                             srow.at[pl.ds(j * 128, 128)], dsem).wait()
            pltpu.async_copy(adW.at[dst2.at[j]],
                             drow.at[pl.ds(j * 128, 128)], dsem).wait()

        @pl.loop(0, 64)
        def _(g):
            evec = g * 4 + epat
            sv = plsc.load_gather(srow, [evec, hpat])
            dv = plsc.load_gather(drow, [evec, hpat])
            e = sv + dv
            e = jnp.maximum(e, 0.0) + 0.2 * jnp.minimum(e, 0.0)
            wv = jnp.exp(e)
            plsc.store_scatter(whm, [hpat, evec // 128, evec % 128], wv)

        pltpu.sync_copy(whm, wW.at[c, :, pl.ds(rb, 2)])

    plsc.subcore_barrier()

    @pl.loop(0, H)
    def _(h):
        zero_denom()
        plsc.subcore_barrier()

        @pl.loop(0, NWIN)
        def _(w):
            base = pl.multiple_of(t * EPT + w * WINE, 256)
            rb = pl.multiple_of(base // 128, 2)
            pltpu.sync_copy(edgeW.at[c, 0, pl.ds(rb, 2)], src2)
            pltpu.sync_copy(edgeW.at[c, 1, pl.ds(rb, 2)], dst2)

            @pl.loop(0, 2)
            def _(r):
                for g in range(8):
                    gix[r, pl.ds(g * 16, 16)] = (
                        src2[r, pl.ds(g * 16, 16)] * 4 + (h + c * (4 * N)))

            for j in range(2):
                pltpu.async_copy(tblW.at[gix.at[j]],
                                 rows.at[pl.ds(j * 128, 128)], dsem).wait()

            pltpu.sync_copy(wW.at[c, h, pl.ds(rb, 2)], w2)
            for j in range(2):
                pltpu.sync_copy(w2.at[j], denom.at[dst2.at[j]], add=True)

            @pl.loop(0, 16)
            def _(g):
                wvec = w2[g // 8, pl.ds((g % 8) * 16, 16)]
                for j in range(16):
                    b = _splat(wvec, j)
                    e = g * 16 + j
                    rows[e, pl.ds(0, 16)] = rows[e, pl.ds(0, 16)] * b
                    rows[e, pl.ds(16, 16)] = rows[e, pl.ds(16, 16)] * b

            for j in range(2):
                pltpu.sync_copy(rows.at[pl.ds(j * 128, 128)],
                                numer.at[dst2.at[j]], add=True)

        plsc.subcore_barrier()
        hc = pl.multiple_of(h * 32, 32)
        for k in range(7):
            ch = t + 16 * k

            @pl.when(ch < 97)
            def _():
                nb = pl.multiple_of(ch * 512, 512)
                pltpu.sync_copy(numer.at[pl.ds(nb, 512)],
                                ftW.at[c, pl.ds(nb, 512), pl.ds(hc, 32)])
                pltpu.sync_copy(denom.at[pl.ds(nb, 512)],
                                denW.at[c, h, pl.ds(nb, 512)])

            @pl.when(ch == 97)
            def _():
                nb = 97 * 512
                pltpu.sync_copy(numer.at[pl.ds(nb, 336)],
                                ftW.at[c, pl.ds(nb, 336), pl.ds(hc, 32)])
                pltpu.sync_copy(denom.at[pl.ds(nb, 336)],
                                denW.at[c, h, pl.ds(nb, 336)])

        plsc.subcore_barrier()
        zero_numer()
        plsc.subcore_barrier()

    @pl.loop(0, 2)
    def _(i):
        rel = c * 2 + i

        @pl.loop(0, H)
        def _(h):
            @pl.loop(0, NWIN)
            def _(w):
                base = pl.multiple_of(t * EPT + w * WINE, 256)
                rb = pl.multiple_of(base // 128, 2)
                pltpu.sync_copy(edgeU.at[rel, 0, pl.ds(rb, 2)], src2)
                pltpu.sync_copy(edgeU.at[rel, 1, pl.ds(rb, 2)], dst2)

                @pl.loop(0, 2)
                def _(r):
                    for g in range(8):
                        gix[r, pl.ds(g * 16, 16)] = (
                            src2[r, pl.ds(g * 16, 16)] * 4
                            + (h + rel * (4 * N)))

                for j in range(2):
                    pltpu.async_copy(tblU.at[gix.at[j]],
                                     rows.at[pl.ds(j * 128, 128)],
                                     dsem).wait()

                for j in range(2):
                    pltpu.sync_copy(rows.at[pl.ds(j * 128, 128)],
                                    numer.at[dst2.at[j]], add=True)

            plsc.subcore_barrier()
            hc = pl.multiple_of(h * 32, 32)
            for k in range(7):
                ch = t + 16 * k

                @pl.when(ch < 97)
                def _():
                    nb = pl.multiple_of(ch * 512, 512)
                    pltpu.sync_copy(numer.at[pl.ds(nb, 512)],
                                    ftU.at[rel, pl.ds(nb, 512),
                                           pl.ds(hc, 32)])

                @pl.when(ch == 97)
                def _():
                    nb = 97 * 512
                    pltpu.sync_copy(numer.at[pl.ds(nb, 336)],
                                    ftU.at[rel, pl.ds(nb, 336),
                                           pl.ds(hc, 32)])

            plsc.subcore_barrier()
            zero_numer()
            plsc.subcore_barrier()


def _make_sc_kernel():
    f32 = jnp.float32
    mesh = plsc.VectorSubcoreMesh(core_axis_name="c", subcore_axis_name="s",
                                  num_cores=2, num_subcores=NT)
    out_type = (
        jax.ShapeDtypeStruct((2, N, HD), f32),
        jax.ShapeDtypeStruct((2, H, EPR, 128), f32),
        jax.ShapeDtypeStruct((2, H, N), f32),
        jax.ShapeDtypeStruct((4, N, HD), f32),
    )
    WR = WINE // 128
    scratch = [
        pltpu.VMEM((WR, 128), jnp.int32),
        pltpu.VMEM((WR, 128), jnp.int32),
        pltpu.VMEM((WR, 128), jnp.int32),
        pltpu.VMEM((WINE, D), f32),
        pltpu.VMEM((WINE, 16), f32),
        pltpu.VMEM((WINE, 16), f32),
        pltpu.VMEM((H, WR, 128), f32),
        pltpu.VMEM((WR, 128), f32),
        pltpu.VMEM((128, D), f32),
        pltpu.VMEM((ZF,), f32),
        pltpu.VMEM_SHARED((NACC, D), f32),
        pltpu.VMEM_SHARED((DLEN,), f32),
        pltpu.SemaphoreType.DMA,
    ]
    return pl.kernel(_sc_edge_kernel, out_type=out_type, mesh=mesh,
                     scratch_types=scratch,
                     compiler_params=pltpu.CompilerParams(
                         use_tc_tiling_on_sc=False,
                         needs_layout_passes=False,
                         internal_scratch_in_bytes=131072))


def _pad_edges(e):
    pad = EP - E
    src = jnp.concatenate([e[0], jnp.zeros((pad,), jnp.int32)])
    dst = jnp.concatenate([e[1], jnp.full((pad,), N, jnp.int32)])
    return jnp.stack([src, dst]).reshape(2, EPR, 128)


def _attn_mat(vec):
    a = jnp.reshape(jnp.eye(H, dtype=vec.dtype)[:, None, :]
                    * vec[0][:, :, None], (HD, H))
    return jnp.pad(a, ((0, 0), (0, 16 - H)))


BN = 1000
GRID = N // BN


def _lin_body(fc, fp, fs, fv, Wcat, bcat, Acs, Acd, Aps, Apd,
              o_crew, o_plane, o_si, o_vto, o_whW, o_tblU, o_asW, o_adW):
    feats = {"c": fc[...], "p": fp[...], "s": fs[...], "v": fv[...]}
    wh = {}
    order = [("crew", "c"), ("plane", "p"), ("c_in", "c"), ("repairing", "c"),
             ("p_in", "p"), ("p_to", "p"), ("repaired_by", "p"),
             ("s_in", "s"), ("s_to", "s"), ("v_to", "v")]
    for i, (nm, fkey) in enumerate(order):
        y = jnp.dot(feats[fkey], Wcat[i], preferred_element_type=jnp.float32)
        wh[nm] = y + bcat[i, 0, :]
    o_crew[...] = wh["crew"]
    o_plane[...] = wh["plane"]
    o_si[...] = wh["s_in"]
    o_vto[...] = wh["v_to"]
    o_whW[0] = wh["c_in"]
    o_whW[1] = wh["p_in"]
    o_tblU[0] = wh["repairing"]
    o_tblU[1] = wh["p_to"]
    o_tblU[2] = wh["repaired_by"]
    o_tblU[3] = wh["s_to"]
    o_asW[0] = jnp.dot(wh["c_in"], Acs[...],
                       preferred_element_type=jnp.float32)
    o_asW[1] = jnp.dot(wh["p_in"], Aps[...],
                       preferred_element_type=jnp.float32)
    o_adW[0] = jnp.dot(wh["s_in"], Acd[...],
                       preferred_element_type=jnp.float32)
    o_adW[1] = jnp.dot(wh["s_in"], Apd[...],
                       preferred_element_type=jnp.float32)


def _combine_body(whc, ftrby, whp, ftrep, whsi, ftci, ftpi, whvt, ftpto,
                  ftsto, dc, dp, o_crew, o_plane, o_state, o_value):
    o_crew[...] = jnp.maximum(whc[...] + ftrby[0], 0.0)
    o_plane[...] = jnp.maximum(whp[...] + ftrep[0], 0.0)
    rc = dc[...]
    rc = jnp.where(rc > 0.0, 1.0 / rc, 0.0)[:, :, None]
    rp = dp[...]
    rp = jnp.where(rp > 0.0, 1.0 / rp, 0.0)[:, :, None]
    ci = (ftci[0].reshape(BN, H, D) * rc).reshape(BN, HD)
    pi = (ftpi[0].reshape(BN, H, D) * rp).reshape(BN, HD)
    o_state[...] = jnp.maximum(whsi[...] + ci + pi, 0.0)
    o_value[...] = jnp.maximum(whvt[...] + ftpto[0] + ftsto[0], 0.0)


def kernel(feat_crew, feat_plane, feat_state, feat_value, edge_c_in,
           edge_repairing, edge_p_in, edge_p_to, edge_repaired_by, edge_s_in,
           edge_s_to, edge_v_to, W_crew, b_crew, W_plane, b_plane, W_c_in,
           b_c_in, W_repairing, b_repairing, W_p_in, b_p_in, W_p_to, b_p_to,
           W_repaired_by, b_repaired_by, W_s_in, b_s_in, W_s_to, b_s_to,
           W_v_to, b_v_to, c_in_src, c_in_dst, p_in_src, p_in_dst):
    f32 = jnp.float32
    Wcat = jnp.stack([W_crew, W_plane, W_c_in, W_repairing, W_p_in, W_p_to,
                      W_repaired_by, W_s_in, W_s_to, W_v_to])
    bcat = jnp.stack([b_crew, b_plane, b_c_in, b_repairing, b_p_in, b_p_to,
                      b_repaired_by, b_s_in, b_s_to, b_v_to])[:, None, :]
    Acs, Acd = _attn_mat(c_in_src), _attn_mat(c_in_dst)
    Aps, Apd = _attn_mat(p_in_src), _attn_mat(p_in_dst)

    row = pl.BlockSpec((BN, HD), lambda i: (i, 0))
    full = lambda s: pl.BlockSpec(s, lambda i: tuple(0 for _ in s))

    lin_outs = pl.pallas_call(
        _lin_body,
        grid=(GRID,),
        in_specs=[row, row, row, row, full((10, DIN, HD)), full((10, 1, HD)),
                  full((HD, 16)), full((HD, 16)), full((HD, 16)),
                  full((HD, 16))],
        out_specs=[row, row, row, row,
                   pl.BlockSpec((2, BN, HD), lambda i: (0, i, 0)),
                   pl.BlockSpec((4, BN, HD), lambda i: (0, i, 0)),
                   pl.BlockSpec((2, BN, 16), lambda i: (0, i, 0)),
                   pl.BlockSpec((2, BN, 16), lambda i: (0, i, 0))],
        out_shape=[jax.ShapeDtypeStruct((N, HD), f32)] * 4
        + [jax.ShapeDtypeStruct((2, N, HD), f32),
           jax.ShapeDtypeStruct((4, N, HD), f32),
           jax.ShapeDtypeStruct((2, N, 16), f32),
           jax.ShapeDtypeStruct((2, N, 16), f32)],
    )(feat_crew, feat_plane, feat_state, feat_value, Wcat, bcat,
      Acs, Acd, Aps, Apd)
    (wh_crew, wh_plane, wh_si, wh_vto, whW, tblU4, asW, adW) = lin_outs

    sc = _make_sc_kernel()
    ftW, _wW, denW, ftU = sc(
        whW.reshape(2 * N * H, D), asW.reshape(2 * N, 16),
        adW.reshape(2 * N, 16), tblU4.reshape(4 * N * H, D),
        jnp.stack([_pad_edges(edge_c_in), _pad_edges(edge_p_in)]),
        jnp.stack([_pad_edges(edge_repairing), _pad_edges(edge_p_to),
                   _pad_edges(edge_repaired_by), _pad_edges(edge_s_to)]))

    def sub(j):
        return pl.BlockSpec((1, BN, HD), lambda i, j=j: (j, i, 0))

    outs = pl.pallas_call(
        _combine_body,
        grid=(GRID,),
        in_specs=[row, sub(2), row, sub(0), row, sub(0), sub(1), row, sub(1),
                  sub(3), pl.BlockSpec((BN, H), lambda i: (i, 0)),
                  pl.BlockSpec((BN, H), lambda i: (i, 0))],
        out_specs=[row] * 4,
        out_shape=[jax.ShapeDtypeStruct((N, HD), f32)] * 4,
    )(wh_crew, ftU, wh_plane, ftU, wh_si, ftW, ftW, wh_vto, ftU, ftU,
      denW[0].T, denW[1].T)
    return tuple(o.reshape(N, H, D) for o in outs)


# speedup vs baseline: 62.4985x; 1.3555x over previous
import jax
import jax.numpy as jnp
from jax import lax
from jax.experimental import pallas as pl
from jax.experimental.pallas import tpu as pltpu
from jax.experimental.pallas import tpu_sc as plsc

H = 4
D = 32
N = 50000
E = 500000
DIN = 128
HD = H * D

NT = 16
WINE = 256
NWIN = 124
EPT = WINE * NWIN
EP = EPT * NT
EPR = EP // 128
NACC = 50048
DLEN = 50176
ZF = DLEN // NT


def _splat(vec, lane):
    idx = jnp.full((16,), lane, dtype=jnp.int32)
    return jnp.take_along_axis(vec, idx, axis=0)


def _sc_edge_kernel(tblW, asW, adW, tblU, edgeW, edgeU,
                    ftW, wW, denW, ftU,
                    src2, dst2, gix, rows, srow, drow, whm, w2,
                    zrow, zflat, numer, denom, dsem):
    c = lax.axis_index("c")
    t = lax.axis_index("s")

    lane = lax.iota(jnp.int32, 16)
    epat = lane // 4
    hpat = lane % 4

    zv = jnp.zeros((16,), jnp.float32)

    @pl.loop(0, 256)
    def _(i):
        zrow[i // 2, pl.ds((i % 2) * 16, 16)] = zv

    @pl.loop(0, ZF // 16)
    def _(i):
        zflat[pl.ds(i * 16, 16)] = zv

    def zero_numer():
        for k in range(25):
            ch = t + 16 * k

            @pl.when(ch < 391)
            def _():
                nb = pl.multiple_of(ch * 128, 128)
                pltpu.sync_copy(zrow, numer.at[pl.ds(nb, 128)])

    def zero_denom():
        pltpu.sync_copy(zflat,
                        denom.at[pl.ds(pl.multiple_of(t * ZF, ZF), ZF)])

    zero_numer()
    plsc.subcore_barrier()

    @pl.loop(0, NWIN)
    def _(w):
        base = pl.multiple_of(t * EPT + w * WINE, 256)
        rb = pl.multiple_of(base // 128, 2)
        d1 = pltpu.async_copy(edgeW.at[c, 0, pl.ds(rb, 2)], src2, dsem)
        d2 = pltpu.async_copy(edgeW.at[c, 1, pl.ds(rb, 2)], dst2, dsem)
        d1.wait()
        d2.wait()

        @pl.loop(0, 2)
        def _(r):
            for g in range(8):
                gix[r, pl.ds(g * 16, 16)] = (
                    src2[r, pl.ds(g * 16, 16)] + c * N)
                dst2[r, pl.ds(g * 16, 16)] = jnp.minimum(
                    dst2[r, pl.ds(g * 16, 16)], N - 1) + c * N

        gds = []
        for j in range(2):
            gds.append(pltpu.async_copy(
                asW.at[gix.at[j]], srow.at[pl.ds(j * 128, 128)], dsem))
            gds.append(pltpu.async_copy(
                adW.at[dst2.at[j]], drow.at[pl.ds(j * 128, 128)], dsem))
        for g_ in gds:
            g_.wait()

        @pl.loop(0, 64)
        def _(g):
            evec = g * 4 + epat
            sv = plsc.load_gather(srow, [evec, hpat])
            dv = plsc.load_gather(drow, [evec, hpat])
            e = sv + dv
            e = jnp.maximum(e, 0.0) + 0.2 * jnp.minimum(e, 0.0)
            wv = jnp.exp(e)
            plsc.store_scatter(whm, [hpat, evec // 128, evec % 128], wv)

        pltpu.sync_copy(whm, wW.at[c, :, pl.ds(rb, 2)])

    plsc.subcore_barrier()

    @pl.loop(0, H)
    def _(h):
        zero_denom()
        plsc.subcore_barrier()

        @pl.loop(0, NWIN)
        def _(w):
            base = pl.multiple_of(t * EPT + w * WINE, 256)
            rb = pl.multiple_of(base // 128, 2)
            d1 = pltpu.async_copy(edgeW.at[c, 0, pl.ds(rb, 2)], src2, dsem)
            d2 = pltpu.async_copy(edgeW.at[c, 1, pl.ds(rb, 2)], dst2, dsem)
            d1.wait()
            d2.wait()

            @pl.loop(0, 2)
            def _(r):
                for g in range(8):
                    gix[r, pl.ds(g * 16, 16)] = (
                        src2[r, pl.ds(g * 16, 16)] * 4 + (h + c * (4 * N)))

            gds = [pltpu.async_copy(wW.at[c, h, pl.ds(rb, 2)], w2, dsem)]
            for j in range(2):
                gds.append(pltpu.async_copy(
                    tblW.at[gix.at[j]], rows.at[pl.ds(j * 128, 128)], dsem))
            for g_ in gds:
                g_.wait()
            sds = [pltpu.async_copy(w2.at[j], denom.at[dst2.at[j]], dsem,
                                    add=True) for j in range(2)]
            for s_ in sds:
                s_.wait()

            @pl.loop(0, 16)
            def _(g):
                wvec = w2[g // 8, pl.ds((g % 8) * 16, 16)]
                for j in range(16):
                    b = _splat(wvec, j)
                    e = g * 16 + j
                    rows[e, pl.ds(0, 16)] = rows[e, pl.ds(0, 16)] * b
                    rows[e, pl.ds(16, 16)] = rows[e, pl.ds(16, 16)] * b

            sds = [pltpu.async_copy(rows.at[pl.ds(j * 128, 128)],
                                    numer.at[dst2.at[j]], dsem, add=True)
                   for j in range(2)]
            for s_ in sds:
                s_.wait()

        plsc.subcore_barrier()
        hc = pl.multiple_of(h * 32, 32)
        for k in range(7):
            ch = t + 16 * k

            @pl.when(ch < 97)
            def _():
                nb = pl.multiple_of(ch * 512, 512)
                pltpu.sync_copy(numer.at[pl.ds(nb, 512)],
                                ftW.at[c, pl.ds(nb, 512), pl.ds(hc, 32)])
                pltpu.sync_copy(denom.at[pl.ds(nb, 512)],
                                denW.at[c, h, pl.ds(nb, 512)])

            @pl.when(ch == 97)
            def _():
                nb = 97 * 512
                pltpu.sync_copy(numer.at[pl.ds(nb, 336)],
                                ftW.at[c, pl.ds(nb, 336), pl.ds(hc, 32)])
                pltpu.sync_copy(denom.at[pl.ds(nb, 336)],
                                denW.at[c, h, pl.ds(nb, 336)])

        plsc.subcore_barrier()
        zero_numer()
        plsc.subcore_barrier()

    @pl.loop(0, 2)
    def _(i):
        rel = c * 2 + i

        @pl.loop(0, H)
        def _(h):
            @pl.loop(0, NWIN)
            def _(w):
                base = pl.multiple_of(t * EPT + w * WINE, 256)
                rb = pl.multiple_of(base // 128, 2)
                d1 = pltpu.async_copy(edgeU.at[rel, 0, pl.ds(rb, 2)],
                                      src2, dsem)
                d2 = pltpu.async_copy(edgeU.at[rel, 1, pl.ds(rb, 2)],
                                      dst2, dsem)
                d1.wait()
                d2.wait()

                @pl.loop(0, 2)
                def _(r):
                    for g in range(8):
                        gix[r, pl.ds(g * 16, 16)] = (
                            src2[r, pl.ds(g * 16, 16)] * 4
                            + (h + rel * (4 * N)))

                gds = [pltpu.async_copy(tblU.at[gix.at[j]],
                                        rows.at[pl.ds(j * 128, 128)], dsem)
                       for j in range(2)]
                for g_ in gds:
                    g_.wait()
                sds = [pltpu.async_copy(rows.at[pl.ds(j * 128, 128)],
                                        numer.at[dst2.at[j]], dsem,
                                        add=True) for j in range(2)]
                for s_ in sds:
                    s_.wait()

            plsc.subcore_barrier()
            hc = pl.multiple_of(h * 32, 32)
            for k in range(7):
                ch = t + 16 * k

                @pl.when(ch < 97)
                def _():
                    nb = pl.multiple_of(ch * 512, 512)
                    pltpu.sync_copy(numer.at[pl.ds(nb, 512)],
                                    ftU.at[rel, pl.ds(nb, 512),
                                           pl.ds(hc, 32)])

                @pl.when(ch == 97)
                def _():
                    nb = 97 * 512
                    pltpu.sync_copy(numer.at[pl.ds(nb, 336)],
                                    ftU.at[rel, pl.ds(nb, 336),
                                           pl.ds(hc, 32)])

            plsc.subcore_barrier()
            zero_numer()
            plsc.subcore_barrier()


def _make_sc_kernel():
    f32 = jnp.float32
    mesh = plsc.VectorSubcoreMesh(core_axis_name="c", subcore_axis_name="s",
                                  num_cores=2, num_subcores=NT)
    out_type = (
        jax.ShapeDtypeStruct((2, N, HD), f32),
        jax.ShapeDtypeStruct((2, H, EPR, 128), f32),
        jax.ShapeDtypeStruct((2, H, N), f32),
        jax.ShapeDtypeStruct((4, N, HD), f32),
    )
    WR = WINE // 128
    scratch = [
        pltpu.VMEM((WR, 128), jnp.int32),
        pltpu.VMEM((WR, 128), jnp.int32),
        pltpu.VMEM((WR, 128), jnp.int32),
        pltpu.VMEM((WINE, D), f32),
        pltpu.VMEM((WINE, 16), f32),
        pltpu.VMEM((WINE, 16), f32),
        pltpu.VMEM((H, WR, 128), f32),
        pltpu.VMEM((WR, 128), f32),
        pltpu.VMEM((128, D), f32),
        pltpu.VMEM((ZF,), f32),
        pltpu.VMEM_SHARED((NACC, D), f32),
        pltpu.VMEM_SHARED((DLEN,), f32),
        pltpu.SemaphoreType.DMA,
    ]
    return pl.kernel(_sc_edge_kernel, out_type=out_type, mesh=mesh,
                     scratch_types=scratch,
                     compiler_params=pltpu.CompilerParams(
                         use_tc_tiling_on_sc=False,
                         needs_layout_passes=False,
                         internal_scratch_in_bytes=131072))


def _pad_edges(e):
    pad = EP - E
    src = jnp.concatenate([e[0], jnp.zeros((pad,), jnp.int32)])
    dst = jnp.concatenate([e[1], jnp.full((pad,), N, jnp.int32)])
    return jnp.stack([src, dst]).reshape(2, EPR, 128)


def _attn_mat(vec):
    a = jnp.reshape(jnp.eye(H, dtype=vec.dtype)[:, None, :]
                    * vec[0][:, :, None], (HD, H))
    return jnp.pad(a, ((0, 0), (0, 16 - H)))


BN = 1000
GRID = N // BN


def _lin_body(fc, fp, fs, fv, Wcat, bcat, Acs, Acd, Aps, Apd,
              o_crew, o_plane, o_si, o_vto, o_whW, o_tblU, o_asW, o_adW):
    feats = {"c": fc[...], "p": fp[...], "s": fs[...], "v": fv[...]}
    wh = {}
    order = [("crew", "c"), ("plane", "p"), ("c_in", "c"), ("repairing", "c"),
             ("p_in", "p"), ("p_to", "p"), ("repaired_by", "p"),
             ("s_in", "s"), ("s_to", "s"), ("v_to", "v")]
    for i, (nm, fkey) in enumerate(order):
        y = jnp.dot(feats[fkey], Wcat[i], preferred_element_type=jnp.float32)
        wh[nm] = y + bcat[i, 0, :]
    o_crew[...] = wh["crew"]
    o_plane[...] = wh["plane"]
    o_si[...] = wh["s_in"]
    o_vto[...] = wh["v_to"]
    o_whW[0] = wh["c_in"]
    o_whW[1] = wh["p_in"]
    o_tblU[0] = wh["repairing"]
    o_tblU[1] = wh["p_to"]
    o_tblU[2] = wh["repaired_by"]
    o_tblU[3] = wh["s_to"]
    o_asW[0] = jnp.dot(wh["c_in"], Acs[...],
                       preferred_element_type=jnp.float32)
    o_asW[1] = jnp.dot(wh["p_in"], Aps[...],
                       preferred_element_type=jnp.float32)
    o_adW[0] = jnp.dot(wh["s_in"], Acd[...],
                       preferred_element_type=jnp.float32)
    o_adW[1] = jnp.dot(wh["s_in"], Apd[...],
                       preferred_element_type=jnp.float32)


def _combine_body(whc, ftrby, whp, ftrep, whsi, ftci, ftpi, whvt, ftpto,
                  ftsto, dc, dp, o_crew, o_plane, o_state, o_value):
    o_crew[...] = jnp.maximum(whc[...] + ftrby[0], 0.0)
    o_plane[...] = jnp.maximum(whp[...] + ftrep[0], 0.0)
    rc = dc[...]
    rc = jnp.where(rc > 0.0, 1.0 / rc, 0.0)[:, :, None]
    rp = dp[...]
    rp = jnp.where(rp > 0.0, 1.0 / rp, 0.0)[:, :, None]
    ci = (ftci[0].reshape(BN, H, D) * rc).reshape(BN, HD)
    pi = (ftpi[0].reshape(BN, H, D) * rp).reshape(BN, HD)
    o_state[...] = jnp.maximum(whsi[...] + ci + pi, 0.0)
    o_value[...] = jnp.maximum(whvt[...] + ftpto[0] + ftsto[0], 0.0)


def kernel(feat_crew, feat_plane, feat_state, feat_value, edge_c_in,
           edge_repairing, edge_p_in, edge_p_to, edge_repaired_by, edge_s_in,
           edge_s_to, edge_v_to, W_crew, b_crew, W_plane, b_plane, W_c_in,
           b_c_in, W_repairing, b_repairing, W_p_in, b_p_in, W_p_to, b_p_to,
           W_repaired_by, b_repaired_by, W_s_in, b_s_in, W_s_to, b_s_to,
           W_v_to, b_v_to, c_in_src, c_in_dst, p_in_src, p_in_dst):
    f32 = jnp.float32
    Wcat = jnp.stack([W_crew, W_plane, W_c_in, W_repairing, W_p_in, W_p_to,
                      W_repaired_by, W_s_in, W_s_to, W_v_to])
    bcat = jnp.stack([b_crew, b_plane, b_c_in, b_repairing, b_p_in, b_p_to,
                      b_repaired_by, b_s_in, b_s_to, b_v_to])[:, None, :]
    Acs, Acd = _attn_mat(c_in_src), _attn_mat(c_in_dst)
    Aps, Apd = _attn_mat(p_in_src), _attn_mat(p_in_dst)

    row = pl.BlockSpec((BN, HD), lambda i: (i, 0))
    full = lambda s: pl.BlockSpec(s, lambda i: tuple(0 for _ in s))

    lin_outs = pl.pallas_call(
        _lin_body,
        grid=(GRID,),
        in_specs=[row, row, row, row, full((10, DIN, HD)), full((10, 1, HD)),
                  full((HD, 16)), full((HD, 16)), full((HD, 16)),
                  full((HD, 16))],
        out_specs=[row, row, row, row,
                   pl.BlockSpec((2, BN, HD), lambda i: (0, i, 0)),
                   pl.BlockSpec((4, BN, HD), lambda i: (0, i, 0)),
                   pl.BlockSpec((2, BN, 16), lambda i: (0, i, 0)),
                   pl.BlockSpec((2, BN, 16), lambda i: (0, i, 0))],
        out_shape=[jax.ShapeDtypeStruct((N, HD), f32)] * 4
        + [jax.ShapeDtypeStruct((2, N, HD), f32),
           jax.ShapeDtypeStruct((4, N, HD), f32),
           jax.ShapeDtypeStruct((2, N, 16), f32),
           jax.ShapeDtypeStruct((2, N, 16), f32)],
    )(feat_crew, feat_plane, feat_state, feat_value, Wcat, bcat,
      Acs, Acd, Aps, Apd)
    (wh_crew, wh_plane, wh_si, wh_vto, whW, tblU4, asW, adW) = lin_outs

    sc = _make_sc_kernel()
    ftW, _wW, denW, ftU = sc(
        whW.reshape(2 * N * H, D), asW.reshape(2 * N, 16),
        adW.reshape(2 * N, 16), tblU4.reshape(4 * N * H, D),
        jnp.stack([_pad_edges(edge_c_in), _pad_edges(edge_p_in)]),
        jnp.stack([_pad_edges(edge_repairing), _pad_edges(edge_p_to),
                   _pad_edges(edge_repaired_by), _pad_edges(edge_s_to)]))

    def sub(j):
        return pl.BlockSpec((1, BN, HD), lambda i, j=j: (j, i, 0))

    outs = pl.pallas_call(
        _combine_body,
        grid=(GRID,),
        in_specs=[row, sub(2), row, sub(0), row, sub(0), sub(1), row, sub(1),
                  sub(3), pl.BlockSpec((BN, H), lambda i: (i, 0)),
                  pl.BlockSpec((BN, H), lambda i: (i, 0))],
        out_specs=[row] * 4,
        out_shape=[jax.ShapeDtypeStruct((N, HD), f32)] * 4,
    )(wh_crew, ftU, wh_plane, ftU, wh_si, ftW, ftW, wh_vto, ftU, ftU,
      denW[0].T, denW[1].T)
    return tuple(o.reshape(N, H, D) for o in outs)


# speedup vs baseline: 63.2945x; 1.0127x over previous
import jax
import jax.numpy as jnp
from jax import lax
from jax.experimental import pallas as pl
from jax.experimental.pallas import tpu as pltpu
from jax.experimental.pallas import tpu_sc as plsc

H = 4
D = 32
N = 50000
E = 500000
DIN = 128
HD = H * D

NT = 16
WINE = 256
NWIN = 124
EPT = WINE * NWIN
EP = EPT * NT
EPR = EP // 128
NACC = 50048
DLEN = 50176
ZF = DLEN // NT


def _splat(vec, lane):
    idx = jnp.full((16,), lane, dtype=jnp.int32)
    return jnp.take_along_axis(vec, idx, axis=0)


def _sc_edge_kernel(tblW, asW, adW, tblU, edgeW, edgeU,
                    ftW, wW, denW, ftU,
                    src2, dst2, gix, rows, srow, drow, whm, w2,
                    zrow, zflat, numer, denom, dsem):
    c = lax.axis_index("c")
    t = lax.axis_index("s")

    lane = lax.iota(jnp.int32, 16)
    epat = lane // 4
    hpat = lane % 4

    zv = jnp.zeros((16,), jnp.float32)

    @pl.loop(0, 256)
    def _(i):
        zrow[i // 2, pl.ds((i % 2) * 16, 16)] = zv

    @pl.loop(0, ZF // 16)
    def _(i):
        zflat[pl.ds(i * 16, 16)] = zv

    def zero_numer():
        for k in range(25):
            ch = t + 16 * k

            @pl.when(ch < 391)
            def _():
                nb = pl.multiple_of(ch * 128, 128)
                pltpu.sync_copy(zrow, numer.at[pl.ds(nb, 128)])

    def zero_denom():
        pltpu.sync_copy(zflat,
                        denom.at[pl.ds(pl.multiple_of(t * ZF, ZF), ZF)])

    zero_numer()
    plsc.subcore_barrier()

    @pl.loop(0, NWIN)
    def _(w):
        base = pl.multiple_of(t * EPT + w * WINE, 256)
        rb = pl.multiple_of(base // 128, 2)
        d1 = pltpu.async_copy(edgeW.at[c, 0, pl.ds(rb, 2)], src2, dsem)
        d2 = pltpu.async_copy(edgeW.at[c, 1, pl.ds(rb, 2)], dst2, dsem)
        d1.wait()
        d2.wait()

        @pl.loop(0, 2)
        def _(r):
            for g in range(8):
                gix[r, pl.ds(g * 16, 16)] = (
                    src2[r, pl.ds(g * 16, 16)] + c * N)
                dst2[r, pl.ds(g * 16, 16)] = jnp.minimum(
                    dst2[r, pl.ds(g * 16, 16)], N - 1) + c * N

        gds = []
        for j in range(2):
            gds.append(pltpu.async_copy(
                asW.at[gix.at[j]], srow.at[pl.ds(j * 128, 128)], dsem))
            gds.append(pltpu.async_copy(
                adW.at[dst2.at[j]], drow.at[pl.ds(j * 128, 128)], dsem))
        for g_ in gds:
            g_.wait()

        @pl.loop(0, 64)
        def _(g):
            evec = g * 4 + epat
            sv = plsc.load_gather(srow, [evec, hpat])
            dv = plsc.load_gather(drow, [evec, hpat])
            e = sv + dv
            e = jnp.maximum(e, 0.0) + 0.2 * jnp.minimum(e, 0.0)
            wv = jnp.exp(e)
            plsc.store_scatter(whm, [hpat, evec // 128, evec % 128], wv)

        pltpu.sync_copy(whm, wW.at[c, :, pl.ds(rb, 2)])

    plsc.subcore_barrier()

    @pl.loop(0, H)
    def _(h):
        zero_denom()
        plsc.subcore_barrier()

        @pl.loop(0, NWIN)
        def _(w):
            base = pl.multiple_of(t * EPT + w * WINE, 256)
            rb = pl.multiple_of(base // 128, 2)
            d1 = pltpu.async_copy(edgeW.at[c, 0, pl.ds(rb, 2)], src2, dsem)
            d2 = pltpu.async_copy(edgeW.at[c, 1, pl.ds(rb, 2)], dst2, dsem)
            d1.wait()
            d2.wait()

            @pl.loop(0, 2)
            def _(r):
                for g in range(8):
                    gix[r, pl.ds(g * 16, 16)] = (
                        src2[r, pl.ds(g * 16, 16)] * 4 + (h + c * (4 * N)))

            gds = [pltpu.async_copy(wW.at[c, h, pl.ds(rb, 2)], w2, dsem)]
            for j in range(2):
                gds.append(pltpu.async_copy(
                    tblW.at[gix.at[j]], rows.at[pl.ds(j * 128, 128)], dsem))
            for g_ in gds:
                g_.wait()
            sds = [pltpu.async_copy(w2.at[j], denom.at[dst2.at[j]], dsem,
                                    add=True) for j in range(2)]

            @pl.loop(0, 16)
            def _(g):
                wvec = w2[g // 8, pl.ds((g % 8) * 16, 16)]
                for j in range(16):
                    b = _splat(wvec, j)
                    e = g * 16 + j
                    rows[e, pl.ds(0, 16)] = rows[e, pl.ds(0, 16)] * b
                    rows[e, pl.ds(16, 16)] = rows[e, pl.ds(16, 16)] * b

            for s_ in sds:
                s_.wait()

            sds = [pltpu.async_copy(rows.at[pl.ds(j * 128, 128)],
                                    numer.at[dst2.at[j]], dsem, add=True)
                   for j in range(2)]
            for s_ in sds:
                s_.wait()

        plsc.subcore_barrier()
        hc = pl.multiple_of(h * 32, 32)
        for k in range(7):
            ch = t + 16 * k

            @pl.when(ch < 97)
            def _():
                nb = pl.multiple_of(ch * 512, 512)
                pltpu.sync_copy(numer.at[pl.ds(nb, 512)],
                                ftW.at[c, pl.ds(nb, 512), pl.ds(hc, 32)])
                pltpu.sync_copy(denom.at[pl.ds(nb, 512)],
                                denW.at[c, h, pl.ds(nb, 512)])

            @pl.when(ch == 97)
            def _():
                nb = 97 * 512
                pltpu.sync_copy(numer.at[pl.ds(nb, 336)],
                                ftW.at[c, pl.ds(nb, 336), pl.ds(hc, 32)])
                pltpu.sync_copy(denom.at[pl.ds(nb, 336)],
                                denW.at[c, h, pl.ds(nb, 336)])

        plsc.subcore_barrier()
        zero_numer()
        plsc.subcore_barrier()

    @pl.loop(0, 2)
    def _(i):
        rel = c * 2 + i

        @pl.loop(0, H)
        def _(h):
            @pl.loop(0, NWIN)
            def _(w):
                base = pl.multiple_of(t * EPT + w * WINE, 256)
                rb = pl.multiple_of(base // 128, 2)
                d1 = pltpu.async_copy(edgeU.at[rel, 0, pl.ds(rb, 2)],
                                      src2, dsem)
                d2 = pltpu.async_copy(edgeU.at[rel, 1, pl.ds(rb, 2)],
                                      dst2, dsem)
                d1.wait()
                d2.wait()

                @pl.loop(0, 2)
                def _(r):
                    for g in range(8):
                        gix[r, pl.ds(g * 16, 16)] = (
                            src2[r, pl.ds(g * 16, 16)] * 4
                            + (h + rel * (4 * N)))

                gds = [pltpu.async_copy(tblU.at[gix.at[j]],
                                        rows.at[pl.ds(j * 128, 128)], dsem)
                       for j in range(2)]
                for g_ in gds:
                    g_.wait()
                sds = [pltpu.async_copy(rows.at[pl.ds(j * 128, 128)],
                                        numer.at[dst2.at[j]], dsem,
                                        add=True) for j in range(2)]
                for s_ in sds:
                    s_.wait()

            plsc.subcore_barrier()
            hc = pl.multiple_of(h * 32, 32)
            for k in range(7):
                ch = t + 16 * k

                @pl.when(ch < 97)
                def _():
                    nb = pl.multiple_of(ch * 512, 512)
                    pltpu.sync_copy(numer.at[pl.ds(nb, 512)],
                                    ftU.at[rel, pl.ds(nb, 512),
                                           pl.ds(hc, 32)])

                @pl.when(ch == 97)
                def _():
                    nb = 97 * 512
                    pltpu.sync_copy(numer.at[pl.ds(nb, 336)],
                                    ftU.at[rel, pl.ds(nb, 336),
                                           pl.ds(hc, 32)])

            plsc.subcore_barrier()
            zero_numer()
            plsc.subcore_barrier()


def _make_sc_kernel():
    f32 = jnp.float32
    mesh = plsc.VectorSubcoreMesh(core_axis_name="c", subcore_axis_name="s",
                                  num_cores=2, num_subcores=NT)
    out_type = (
        jax.ShapeDtypeStruct((2, N, HD), f32),
        jax.ShapeDtypeStruct((2, H, EPR, 128), f32),
        jax.ShapeDtypeStruct((2, H, N), f32),
        jax.ShapeDtypeStruct((4, N, HD), f32),
    )
    WR = WINE // 128
    scratch = [
        pltpu.VMEM((WR, 128), jnp.int32),
        pltpu.VMEM((WR, 128), jnp.int32),
        pltpu.VMEM((WR, 128), jnp.int32),
        pltpu.VMEM((WINE, D), f32),
        pltpu.VMEM((WINE, 16), f32),
        pltpu.VMEM((WINE, 16), f32),
        pltpu.VMEM((H, WR, 128), f32),
        pltpu.VMEM((WR, 128), f32),
        pltpu.VMEM((128, D), f32),
        pltpu.VMEM((ZF,), f32),
        pltpu.VMEM_SHARED((NACC, D), f32),
        pltpu.VMEM_SHARED((DLEN,), f32),
        pltpu.SemaphoreType.DMA,
    ]
    return pl.kernel(_sc_edge_kernel, out_type=out_type, mesh=mesh,
                     scratch_types=scratch,
                     compiler_params=pltpu.CompilerParams(
                         use_tc_tiling_on_sc=False,
                         needs_layout_passes=False,
                         internal_scratch_in_bytes=131072))


def _pad_edges(e):
    pad = EP - E
    src = jnp.concatenate([e[0], jnp.zeros((pad,), jnp.int32)])
    dst = jnp.concatenate([e[1], jnp.full((pad,), N, jnp.int32)])
    return jnp.stack([src, dst]).reshape(2, EPR, 128)


def _attn_mat(vec):
    a = jnp.reshape(jnp.eye(H, dtype=vec.dtype)[:, None, :]
                    * vec[0][:, :, None], (HD, H))
    return jnp.pad(a, ((0, 0), (0, 16 - H)))


BN = 1000
GRID = N // BN


def _lin_body(fc, fp, fs, fv, Wcat, bcat, Acs, Acd, Aps, Apd,
              o_crew, o_plane, o_si, o_vto, o_whW, o_tblU, o_asW, o_adW):
    feats = {"c": fc[...], "p": fp[...], "s": fs[...], "v": fv[...]}
    wh = {}
    order = [("crew", "c"), ("plane", "p"), ("c_in", "c"), ("repairing", "c"),
             ("p_in", "p"), ("p_to", "p"), ("repaired_by", "p"),
             ("s_in", "s"), ("s_to", "s"), ("v_to", "v")]
    for i, (nm, fkey) in enumerate(order):
        y = jnp.dot(feats[fkey], Wcat[i], preferred_element_type=jnp.float32)
        wh[nm] = y + bcat[i, 0, :]
    o_crew[...] = wh["crew"]
    o_plane[...] = wh["plane"]
    o_si[...] = wh["s_in"]
    o_vto[...] = wh["v_to"]
    o_whW[0] = wh["c_in"]
    o_whW[1] = wh["p_in"]
    o_tblU[0] = wh["repairing"]
    o_tblU[1] = wh["p_to"]
    o_tblU[2] = wh["repaired_by"]
    o_tblU[3] = wh["s_to"]
    o_asW[0] = jnp.dot(wh["c_in"], Acs[...],
                       preferred_element_type=jnp.float32)
    o_asW[1] = jnp.dot(wh["p_in"], Aps[...],
                       preferred_element_type=jnp.float32)
    o_adW[0] = jnp.dot(wh["s_in"], Acd[...],
                       preferred_element_type=jnp.float32)
    o_adW[1] = jnp.dot(wh["s_in"], Apd[...],
                       preferred_element_type=jnp.float32)


def _combine_body(whc, ftrby, whp, ftrep, whsi, ftci, ftpi, whvt, ftpto,
                  ftsto, dc, dp, o_crew, o_plane, o_state, o_value):
    o_crew[...] = jnp.maximum(whc[...] + ftrby[0], 0.0)
    o_plane[...] = jnp.maximum(whp[...] + ftrep[0], 0.0)
    rc = dc[...]
    rc = jnp.where(rc > 0.0, 1.0 / rc, 0.0)[:, :, None]
    rp = dp[...]
    rp = jnp.where(rp > 0.0, 1.0 / rp, 0.0)[:, :, None]
    ci = (ftci[0].reshape(BN, H, D) * rc).reshape(BN, HD)
    pi = (ftpi[0].reshape(BN, H, D) * rp).reshape(BN, HD)
    o_state[...] = jnp.maximum(whsi[...] + ci + pi, 0.0)
    o_value[...] = jnp.maximum(whvt[...] + ftpto[0] + ftsto[0], 0.0)


def kernel(feat_crew, feat_plane, feat_state, feat_value, edge_c_in,
           edge_repairing, edge_p_in, edge_p_to, edge_repaired_by, edge_s_in,
           edge_s_to, edge_v_to, W_crew, b_crew, W_plane, b_plane, W_c_in,
           b_c_in, W_repairing, b_repairing, W_p_in, b_p_in, W_p_to, b_p_to,
           W_repaired_by, b_repaired_by, W_s_in, b_s_in, W_s_to, b_s_to,
           W_v_to, b_v_to, c_in_src, c_in_dst, p_in_src, p_in_dst):
    f32 = jnp.float32
    Wcat = jnp.stack([W_crew, W_plane, W_c_in, W_repairing, W_p_in, W_p_to,
                      W_repaired_by, W_s_in, W_s_to, W_v_to])
    bcat = jnp.stack([b_crew, b_plane, b_c_in, b_repairing, b_p_in, b_p_to,
                      b_repaired_by, b_s_in, b_s_to, b_v_to])[:, None, :]
    Acs, Acd = _attn_mat(c_in_src), _attn_mat(c_in_dst)
    Aps, Apd = _attn_mat(p_in_src), _attn_mat(p_in_dst)

    row = pl.BlockSpec((BN, HD), lambda i: (i, 0))
    full = lambda s: pl.BlockSpec(s, lambda i: tuple(0 for _ in s))

    lin_outs = pl.pallas_call(
        _lin_body,
        grid=(GRID,),
        in_specs=[row, row, row, row, full((10, DIN, HD)), full((10, 1, HD)),
                  full((HD, 16)), full((HD, 16)), full((HD, 16)),
                  full((HD, 16))],
        out_specs=[row, row, row, row,
                   pl.BlockSpec((2, BN, HD), lambda i: (0, i, 0)),
                   pl.BlockSpec((4, BN, HD), lambda i: (0, i, 0)),
                   pl.BlockSpec((2, BN, 16), lambda i: (0, i, 0)),
                   pl.BlockSpec((2, BN, 16), lambda i: (0, i, 0))],
        out_shape=[jax.ShapeDtypeStruct((N, HD), f32)] * 4
        + [jax.ShapeDtypeStruct((2, N, HD), f32),
           jax.ShapeDtypeStruct((4, N, HD), f32),
           jax.ShapeDtypeStruct((2, N, 16), f32),
           jax.ShapeDtypeStruct((2, N, 16), f32)],
    )(feat_crew, feat_plane, feat_state, feat_value, Wcat, bcat,
      Acs, Acd, Aps, Apd)
    (wh_crew, wh_plane, wh_si, wh_vto, whW, tblU4, asW, adW) = lin_outs

    sc = _make_sc_kernel()
    ftW, _wW, denW, ftU = sc(
        whW.reshape(2 * N * H, D), asW.reshape(2 * N, 16),
        adW.reshape(2 * N, 16), tblU4.reshape(4 * N * H, D),
        jnp.stack([_pad_edges(edge_c_in), _pad_edges(edge_p_in)]),
        jnp.stack([_pad_edges(edge_repairing), _pad_edges(edge_p_to),
                   _pad_edges(edge_repaired_by), _pad_edges(edge_s_to)]))

    def sub(j):
        return pl.BlockSpec((1, BN, HD), lambda i, j=j: (j, i, 0))

    outs = pl.pallas_call(
        _combine_body,
        grid=(GRID,),
        in_specs=[row, sub(2), row, sub(0), row, sub(0), sub(1), row, sub(1),
                  sub(3), pl.BlockSpec((BN, H), lambda i: (i, 0)),
                  pl.BlockSpec((BN, H), lambda i: (i, 0))],
        out_specs=[row] * 4,
        out_shape=[jax.ShapeDtypeStruct((N, HD), f32)] * 4,
    )(wh_crew, ftU, wh_plane, ftU, wh_si, ftW, ftW, wh_vto, ftU, ftU,
      denW[0].T, denW[1].T)
    return tuple(o.reshape(N, H, D) for o in outs)
